# baseline (speedup 1.0000x reference)
import jax
import jax.numpy as jnp
from jax.experimental import pallas as pl

D, H, DFF, K, S = 512, 8, 2048, 4, 4
DH = D // H
SHAPES = [(56, 56), (28, 28), (14, 14), (7, 7)]


def _ln(x, g, b, eps=1e-5):
    mu = jnp.mean(x, axis=-1, keepdims=True)
    var = jnp.mean(jnp.square(x - mu), axis=-1, keepdims=True)
    return (x - mu) * jax.lax.rsqrt(var + eps) * g + b


def _bilinear(value, loc):
    b, hl, wl, h, dh = value.shape
    x = loc[..., 0] * wl - 0.5
    y = loc[..., 1] * hl - 0.5
    x0f = jnp.floor(x); y0f = jnp.floor(y)
    wx1 = x - x0f; wx0 = 1.0 - wx1
    wy1 = y - y0f; wy0 = 1.0 - wy1
    x0 = x0f.astype(jnp.int32); y0 = y0f.astype(jnp.int32)
    vflat = value.transpose(0, 3, 1, 2, 4).reshape(b, h, hl * wl, dh)

    def corner(xi, yi, w):
        valid = ((xi >= 0) & (xi < wl) & (yi >= 0) & (yi < hl)).astype(value.dtype)
        idx = jnp.clip(yi, 0, hl - 1) * wl + jnp.clip(xi, 0, wl - 1)
        idx = idx.transpose(0, 2, 1, 3)
        lq, kk = idx.shape[2], idx.shape[3]
        g = jnp.take_along_axis(vflat, idx.reshape(b, h, lq * kk)[..., None], axis=2)
        g = g.reshape(b, h, lq, kk, dh)
        wv = (w * valid).transpose(0, 2, 1, 3)[..., None]
        return g * wv

    out = (corner(x0, y0, wx0 * wy0) + corner(x0 + 1, y0, wx1 * wy0)
           + corner(x0, y0 + 1, wx0 * wy1) + corner(x0 + 1, y0 + 1, wx1 * wy1))
    return out.transpose(0, 2, 1, 3, 4)


def _touch_kernel(x_ref, o_ref):
    o_ref[...] = x_ref[...]


def kernel(src0, src1, src2, src3, ref0, ref1, ref2, ref3, Wv, bv, Woff, boff, Wattn, battn, Wo, bo, W1, b1, W2, b2, g1, be1, g2, be2):
    srcs = [src0, src1, src2, src3]
    refs = [ref0, ref1, ref2, ref3]
    values = [(s @ Wv + bv).reshape(s.shape[0], s.shape[1], s.shape[2], H, DH) for s in srcs]
    outs = []
    for src_l, ref_l in zip(srcs, refs):
        b = src_l.shape[0]
        q = src_l.reshape(b, -1, D)
        ref = ref_l.reshape(b, -1, 2)
        lq = q.shape[1]
        off = (q @ Woff + boff).reshape(b, lq, H, S, K, 2)
        attn = jax.nn.softmax((q @ Wattn + battn).reshape(b, lq, H, S * K), axis=-1)
        attn = attn.reshape(b, lq, H, S, K)
        acc = jnp.zeros((b, lq, H, DH), q.dtype)
        for li, (v, (hl, wl)) in enumerate(zip(values, SHAPES)):
            scale = jnp.array([wl, hl], q.dtype)
            loc = ref[:, :, None, None, :] + off[:, :, :, li] / scale
            samp = _bilinear(v, loc)
            acc = acc + jnp.einsum('blhk,blhkd->blhd', attn[:, :, :, li], samp)
        a = acc.reshape(b, lq, D) @ Wo + bo
        x = _ln(q + a, g1, be1)
        f = jax.nn.relu(x @ W1 + b1) @ W2 + b2
        x = _ln(x + f, g2, be2)
        outs.append(x)
    out = jnp.concatenate(outs, axis=1)
    z = pl.pallas_call(
        _touch_kernel,
        out_shape=jax.ShapeDtypeStruct((8, 128), jnp.float32),
    )(jnp.zeros((8, 128), jnp.float32))
    return out + z[0, 0]


# speedup vs baseline: 32.8417x; 32.8417x over previous
import functools

import jax
import jax.numpy as jnp
from jax import lax
from jax.experimental import pallas as pl
from jax.experimental.pallas import tpu as pltpu

D, H, DFF, KPTS, NS = 512, 8, 2048, 4, 4
DH = D // H
SHAPES = [(56, 56), (28, 28), (14, 14), (7, 7)]
HWS = [h * w for h, w in SHAPES]
SOFFS = [0, 3136, 3920, 4116]
L = 4165
LP = 4224
RPH = 4168
NROWS = H * RPH
TB1 = 256
TB2 = 8
TB3 = 512
NSAMP = H * NS * KPTS


def _k1_body(x_ref, ref_ref, wv_ref, bv_ref, woff_ref, boff_ref, wattn_ref,
             battn_ref, v_ref, a0_ref, a1_ref, b0_ref, b1_ref, idx_ref):
    x = x_ref[...]
    v_ref[...] = jnp.dot(x, wv_ref[...], preferred_element_type=jnp.float32) + bv_ref[...]
    off = jnp.dot(x, woff_ref[...], preferred_element_type=jnp.float32) + boff_ref[...]
    logits = jnp.dot(x, wattn_ref[...], preferred_element_type=jnp.float32) + battn_ref[...]
    m = jnp.max(logits, axis=-1, keepdims=True)
    e = jnp.exp(logits - m)
    r = lax.broadcasted_iota(jnp.int32, (NSAMP, NSAMP), 0) // 16
    c = lax.broadcasted_iota(jnp.int32, (NSAMP, NSAMP), 1) // 16
    seg = (r == c).astype(jnp.float32)
    denom = jnp.dot(e, seg, preferred_element_type=jnp.float32)
    attn = e / denom

    lane = lax.broadcasted_iota(jnp.int32, (TB1, NSAMP), 1)
    li = (lane % 16) // 4
    sci = jnp.where(li == 0, 56, jnp.where(li == 1, 28, jnp.where(li == 2, 14, 7)))
    soff = jnp.where(li == 0, SOFFS[0], jnp.where(li == 1, SOFFS[1],
                     jnp.where(li == 2, SOFFS[2], SOFFS[3])))
    hoff = (lane // 16) * RPH
    sc = sci.astype(jnp.float32)

    offx = off[:, :NSAMP]
    offy = off[:, NSAMP:]
    refx = ref_ref[:, 0:1]
    refy = ref_ref[:, 1:2]

    xf = (refx + offx / sc) * sc - 0.5
    x0f = jnp.floor(xf)
    wx1 = xf - x0f
    wx0 = 1.0 - wx1
    x0 = x0f.astype(jnp.int32)
    bx = jnp.clip(x0, 0, sci - 2)
    a0_ref[...] = attn * (wx0 * (bx == x0) + wx1 * (bx == x0 + 1))
    a1_ref[...] = attn * (wx0 * (bx + 1 == x0) + wx1 * (bx + 1 == x0 + 1))

    yf = (refy + offy / sc) * sc - 0.5
    y0f = jnp.floor(yf)
    wy1 = yf - y0f
    wy0 = 1.0 - wy1
    y0 = y0f.astype(jnp.int32)
    by = jnp.clip(y0, 0, sci - 2)
    b0_ref[...] = wy0 * (by == y0) + wy1 * (by == y0 + 1)
    b1_ref[...] = wy0 * (by + 1 == y0) + wy1 * (by + 1 == y0 + 1)

    idx_ref[...] = hoff + soff + by * sci + bx


def _k2_body(idx_ref, a0_ref, a1_ref, b0_ref, b1_ref, tbl_ref, out_ref, g0, g1):
    a0 = a0_ref[...]
    a1 = a1_ref[...]
    b0 = b0_ref[...]
    b1 = b1_ref[...]
    dn = (((0,), (0,)), ((), ()))
    for t in range(TB2):
        for s in range(NSAMP):
            i = idx_ref[t, s]
            g0[s, :] = tbl_ref[i, 0, :]
            g1[s, :] = tbl_ref[i + 1, 0, :]
        oh = (lax.broadcasted_iota(jnp.int32, (TB2, NSAMP), 0) == t).astype(jnp.float32)
        oh64 = (lax.broadcasted_iota(jnp.int32, (TB2, DH), 0) == t).astype(jnp.float32)
        a0c = lax.dot_general(a0, oh, dn, preferred_element_type=jnp.float32)
        a1c = lax.dot_general(a1, oh, dn, preferred_element_type=jnp.float32)
        b0c = lax.dot_general(b0, oh64, dn, preferred_element_type=jnp.float32)
        b1c = lax.dot_general(b1, oh64, dn, preferred_element_type=jnp.float32)
        p = g0[...] * a0c + g1[...] * a1c
        rr = p[:, :DH] * b0c + p[:, DH:] * b1c
        out_ref[t * H:(t + 1) * H, :] = rr.reshape(H, 16, DH).sum(axis=1)


def _ln(x, g, b, eps=1e-5):
    mu = jnp.mean(x, axis=-1, keepdims=True)
    var = jnp.mean(jnp.square(x - mu), axis=-1, keepdims=True)
    return (x - mu) * lax.rsqrt(var + eps) * g + b


def _k3_body(x_ref, acc_ref, wo_ref, bo_ref, w1_ref, b1_ref, w2_ref, b2_ref,
             g1_ref, be1_ref, g2_ref, be2_ref, out_ref):
    q = x_ref[...]
    a = jnp.dot(acc_ref[...], wo_ref[...], preferred_element_type=jnp.float32) + bo_ref[...]
    x = _ln(q + a, g1_ref[...], be1_ref[...])
    f = jnp.maximum(jnp.dot(x, w1_ref[...], preferred_element_type=jnp.float32) + b1_ref[...], 0.0)
    f = jnp.dot(f, w2_ref[...], preferred_element_type=jnp.float32) + b2_ref[...]
    out_ref[...] = _ln(x + f, g2_ref[...], be2_ref[...])


def _full(shape):
    return pl.BlockSpec(shape, lambda *_: tuple(0 for _ in shape))


@jax.jit
def kernel(src0, src1, src2, src3, ref0, ref1, ref2, ref3, Wv, bv, Woff, boff,
           Wattn, battn, Wo, bo, W1, b1, W2, b2, g1, be1, g2, be2):
    B = src0.shape[0]
    srcs = [src0, src1, src2, src3]
    refs = [ref0, ref1, ref2, ref3]
    x_all = jnp.concatenate([s.reshape(B, -1, D) for s in srcs], axis=1)
    x_all = jnp.pad(x_all, ((0, 0), (0, LP - L), (0, 0)))
    ref_all = jnp.concatenate([r.reshape(B, -1, 2) for r in refs], axis=1)
    ref_all = jnp.pad(ref_all, ((0, 0), (0, LP - L), (0, 0)))
    bl = B * LP
    x_flat = x_all.reshape(bl, D)
    ref_flat = ref_all.reshape(bl, 2)

    woff_p = Woff.reshape(D, H, NS, KPTS, 2).transpose(0, 4, 1, 2, 3).reshape(D, 2 * NSAMP)
    boff_p = boff.reshape(H, NS, KPTS, 2).transpose(3, 0, 1, 2).reshape(1, 2 * NSAMP)

    n1 = bl // TB1
    tok = lambda i: (i, 0)
    v, ca0, ca1, cb0, cb1, idx = pl.pallas_call(
        _k1_body,
        grid=(n1,),
        in_specs=[
            pl.BlockSpec((TB1, D), tok),
            pl.BlockSpec((TB1, 2), tok),
            _full((D, D)), _full((1, D)),
            _full((D, 2 * NSAMP)), _full((1, 2 * NSAMP)),
            _full((D, NSAMP)), _full((1, NSAMP)),
        ],
        out_specs=[
            pl.BlockSpec((TB1, D), tok),
            pl.BlockSpec((TB1, NSAMP), tok), pl.BlockSpec((TB1, NSAMP), tok),
            pl.BlockSpec((TB1, NSAMP), tok), pl.BlockSpec((TB1, NSAMP), tok),
            pl.BlockSpec((TB1, NSAMP), tok),
        ],
        out_shape=[
            jax.ShapeDtypeStruct((bl, D), jnp.float32),
            jax.ShapeDtypeStruct((bl, NSAMP), jnp.float32),
            jax.ShapeDtypeStruct((bl, NSAMP), jnp.float32),
            jax.ShapeDtypeStruct((bl, NSAMP), jnp.float32),
            jax.ShapeDtypeStruct((bl, NSAMP), jnp.float32),
            jax.ShapeDtypeStruct((bl, NSAMP), jnp.int32),
        ],
        compiler_params=pltpu.CompilerParams(dimension_semantics=("parallel",)),
    )(x_flat, ref_flat, Wv, bv.reshape(1, D), woff_p, boff_p, Wattn,
      battn.reshape(1, NSAMP))

    v4 = v.reshape(B, LP, H, DH)
    parts = []
    for (hl, wl), hw, so in zip(SHAPES, HWS, SOFFS):
        seg = v4[:, so:so + hw]
        shifted = jnp.concatenate(
            [seg[:, wl:], jnp.zeros((B, wl, H, DH), jnp.float32)], axis=1)
        parts.append(jnp.concatenate([seg, shifted], axis=-1))
    tbl = jnp.concatenate(parts, axis=1)
    tbl = jnp.pad(tbl, ((0, 0), (0, RPH - L), (0, 0), (0, 0)))
    tbl = tbl.transpose(0, 2, 1, 3).reshape(B, NROWS, 1, 2 * DH)

    n2 = LP // TB2
    acc = pl.pallas_call(
        _k2_body,
        grid=(B, n2),
        in_specs=[
            pl.BlockSpec((None, TB2, NSAMP), lambda b, t: (b, t, 0),
                         memory_space=pltpu.SMEM),
            pl.BlockSpec((None, TB2, NSAMP), lambda b, t: (b, t, 0)),
            pl.BlockSpec((None, TB2, NSAMP), lambda b, t: (b, t, 0)),
            pl.BlockSpec((None, TB2, NSAMP), lambda b, t: (b, t, 0)),
            pl.BlockSpec((None, TB2, NSAMP), lambda b, t: (b, t, 0)),
            pl.BlockSpec((None, NROWS, 1, 2 * DH), lambda b, t: (b, 0, 0, 0)),
        ],
        out_specs=pl.BlockSpec((None, TB2 * H, DH), lambda b, t: (b, t, 0)),
        out_shape=jax.ShapeDtypeStruct((B, LP * H, DH), jnp.float32),
        scratch_shapes=[pltpu.VMEM((NSAMP, 2 * DH), jnp.float32),
                        pltpu.VMEM((NSAMP, 2 * DH), jnp.float32)],
        compiler_params=pltpu.CompilerParams(
            dimension_semantics=("parallel", "arbitrary")),
    )(idx.reshape(B, LP, NSAMP), ca0.reshape(B, LP, NSAMP),
      ca1.reshape(B, LP, NSAMP), cb0.reshape(B, LP, NSAMP),
      cb1.reshape(B, LP, NSAMP), tbl)

    acc_flat = acc.reshape(B, LP, D).reshape(bl, D)

    n3 = bl // TB3
    out = pl.pallas_call(
        _k3_body,
        grid=(n3,),
        in_specs=[
            pl.BlockSpec((TB3, D), tok),
            pl.BlockSpec((TB3, D), tok),
            _full((D, D)), _full((1, D)),
            _full((D, DFF)), _full((1, DFF)),
            _full((DFF, D)), _full((1, D)),
            _full((1, D)), _full((1, D)), _full((1, D)), _full((1, D)),
        ],
        out_specs=pl.BlockSpec((TB3, D), tok),
        out_shape=jax.ShapeDtypeStruct((bl, D), jnp.float32),
        compiler_params=pltpu.CompilerParams(dimension_semantics=("parallel",)),
    )(x_flat, acc_flat, Wo, bo.reshape(1, D), W1, b1.reshape(1, DFF), W2,
      b2.reshape(1, D), g1.reshape(1, D), be1.reshape(1, D), g2.reshape(1, D),
      be2.reshape(1, D))

    return out.reshape(B, LP, D)[:, :L]
